# baseline (speedup 1.0000x reference)
import functools

import jax
import jax.numpy as jnp
from jax import lax
from jax.experimental import pallas as pl
from jax.experimental.pallas import tpu as pltpu
from jax.experimental.pallas import tpu_sc as plsc

VOCAB = 32000
D_MODEL = 4096
N_TOKENS = 4 * 2048

_NUM_CORES = 2
_NUM_SUBCORES = 16
_NW = _NUM_CORES * _NUM_SUBCORES
_ROWS_PER_W = N_TOKENS // _NW
_CHUNK = 4
_N_CHUNKS = _ROWS_PER_W // _CHUNK
_NBUF = 4

_mesh = plsc.VectorSubcoreMesh(core_axis_name="c", subcore_axis_name="s")


@functools.partial(
    pl.kernel,
    mesh=_mesh,
    out_type=jax.ShapeDtypeStruct((N_TOKENS, D_MODEL), jnp.float32),
    scratch_types=[
        pltpu.VMEM((_N_CHUNKS, _CHUNK), jnp.int32),
        pltpu.VMEM((_CHUNK, D_MODEL), jnp.float32),
        pltpu.VMEM((_CHUNK, D_MODEL), jnp.float32),
        pltpu.VMEM((_CHUNK, D_MODEL), jnp.float32),
        pltpu.VMEM((_CHUNK, D_MODEL), jnp.float32),
        pltpu.SemaphoreType.DMA,
        pltpu.SemaphoreType.DMA,
        pltpu.SemaphoreType.DMA,
        pltpu.SemaphoreType.DMA,
        pltpu.SemaphoreType.DMA,
        pltpu.SemaphoreType.DMA,
        pltpu.SemaphoreType.DMA,
        pltpu.SemaphoreType.DMA,
    ],
)
def _embed_gather(
    x_hbm, tab_hbm, out_hbm, idx_v, b0, b1, b2, b3, g0, g1, g2, g3, o0, o1, o2, o3
):
    wid = lax.axis_index("s") * _NUM_CORES + lax.axis_index("c")
    base = wid * _ROWS_PER_W
    pltpu.sync_copy(x_hbm.at[wid], idx_v)

    bufs = (b0, b1, b2, b3)
    gsems = (g0, g1, g2, g3)
    osems = (o0, o1, o2, o3)

    def gd(i, b):
        return pltpu.make_async_copy(tab_hbm.at[idx_v.at[i]], bufs[b], gsems[b])

    def wd(i, b):
        return pltpu.make_async_copy(
            bufs[b], out_hbm.at[pl.ds(base + i * _CHUNK, _CHUNK)], osems[b]
        )

    gd(0, 0).start()
    gd(1, 1).start()

    for i in (0, 1):
        gd(i, i).wait()
        wd(i, i).start()
        gd(i + 2, i + 2).start()

    def body(j, carry):
        for r in range(_NBUF):
            i = 2 + j * _NBUF + r
            b = (2 + r) % _NBUF
            nb = r
            gd(i, b).wait()
            wd(i, b).start()
            wd(i - 2, nb).wait()
            gd(i + 2, nb).start()
        return carry

    lax.fori_loop(0, (_N_CHUNKS - 4) // _NBUF, body, 0, unroll=False)

    for i in (_N_CHUNKS - 2, _N_CHUNKS - 1):
        b = i % _NBUF
        gd(i, b).wait()
        wd(i, b).start()
        wd(i - 2, (b + 2) % _NBUF).wait()
    wd(_N_CHUNKS - 2, (_N_CHUNKS - 2) % _NBUF).wait()
    wd(_N_CHUNKS - 1, (_N_CHUNKS - 1) % _NBUF).wait()


def kernel(x, embed_tokens):
    flat = x.reshape(_NW, _N_CHUNKS, _CHUNK).astype(jnp.int32)
    out = _embed_gather(flat, embed_tokens)
    return out.reshape(x.shape[0], x.shape[1], D_MODEL)


# speedup vs baseline: 1.0044x; 1.0044x over previous
import functools

import jax
import jax.numpy as jnp
from jax import lax
from jax.experimental import pallas as pl
from jax.experimental.pallas import tpu as pltpu
from jax.experimental.pallas import tpu_sc as plsc

VOCAB = 32000
D_MODEL = 4096
N_TOKENS = 4 * 2048

_NUM_CORES = 2
_NUM_SUBCORES = 16
_NW = _NUM_CORES * _NUM_SUBCORES
_ROWS_PER_W = N_TOKENS // _NW
_CHUNK = 8
_N_CHUNKS = _ROWS_PER_W // _CHUNK

_mesh = plsc.VectorSubcoreMesh(core_axis_name="c", subcore_axis_name="s")


@functools.partial(
    pl.kernel,
    mesh=_mesh,
    out_type=jax.ShapeDtypeStruct((N_TOKENS, D_MODEL), jnp.float32),
    scratch_types=[
        pltpu.VMEM((_ROWS_PER_W,), jnp.int32),
        pltpu.VMEM((_CHUNK, D_MODEL), jnp.float32),
        pltpu.VMEM((_CHUNK, D_MODEL), jnp.float32),
        pltpu.SemaphoreType.DMA,
        pltpu.SemaphoreType.DMA,
        pltpu.SemaphoreType.DMA,
        pltpu.SemaphoreType.DMA,
    ],
)
def _embed_gather(x_hbm, tab_hbm, out_hbm, idx_v, buf0, buf1, g0, g1, o0, o1):
    wid = lax.axis_index("s") * _NUM_CORES + lax.axis_index("c")
    base = wid * _ROWS_PER_W
    pltpu.sync_copy(x_hbm.at[pl.ds(base, _ROWS_PER_W)], idx_v)

    bufs = (buf0, buf1)
    gsems = (g0, g1)
    osems = (o0, o1)

    def gather_desc(i, b):
        return pltpu.make_async_copy(
            tab_hbm.at[idx_v.at[pl.ds(i * _CHUNK, _CHUNK)]], bufs[b], gsems[b]
        )

    def write_desc(i, b):
        return pltpu.make_async_copy(
            bufs[b], out_hbm.at[pl.ds(base + i * _CHUNK, _CHUNK)], osems[b]
        )

    gather_desc(0, 0).start()
    gather_desc(1, 1).start()

    def body(j, carry):
        for b in range(2):
            i = 2 * j + b
            gather_desc(i, b).wait()
            write_desc(i, b).start()
            write_desc(i, b).wait()
            gather_desc(i + 2, b).start()
        return carry

    lax.fori_loop(0, _N_CHUNKS // 2 - 1, body, 0, unroll=False)

    for b in range(2):
        i = _N_CHUNKS - 2 + b
        gather_desc(i, b).wait()
        write_desc(i, b).start()
        write_desc(i, b).wait()


def kernel(x, embed_tokens):
    flat = x.reshape(-1).astype(jnp.int32)
    out = _embed_gather(flat, embed_tokens)
    return out.reshape(x.shape[0], x.shape[1], D_MODEL)


# speedup vs baseline: 1.0081x; 1.0037x over previous
import functools

import jax
import jax.numpy as jnp
from jax import lax
from jax.experimental import pallas as pl
from jax.experimental.pallas import tpu as pltpu
from jax.experimental.pallas import tpu_sc as plsc

VOCAB = 32000
D_MODEL = 4096
N_TOKENS = 4 * 2048

_NUM_CORES = 2
_NUM_SUBCORES = 16
_NW = _NUM_CORES * _NUM_SUBCORES
_ROWS_PER_W = N_TOKENS // _NW
_CHUNK = 8
_N_CHUNKS = _ROWS_PER_W // _CHUNK

_mesh = plsc.VectorSubcoreMesh(core_axis_name="c", subcore_axis_name="s")


@functools.partial(
    pl.kernel,
    mesh=_mesh,
    out_type=jax.ShapeDtypeStruct((N_TOKENS, D_MODEL), jnp.float32),
    scratch_types=[
        pltpu.VMEM((_ROWS_PER_W,), jnp.int32),
        pltpu.VMEM((_CHUNK, D_MODEL), jnp.float32),
        pltpu.VMEM((_CHUNK, D_MODEL), jnp.float32),
        pltpu.SemaphoreType.DMA,
        pltpu.SemaphoreType.DMA,
        pltpu.SemaphoreType.DMA,
        pltpu.SemaphoreType.DMA,
    ],
)
def _embed_gather(x_hbm, tab_hbm, out_hbm, idx_v, buf0, buf1, g0, g1, o0, o1):
    wid = lax.axis_index("s") * _NUM_CORES + lax.axis_index("c")
    base = wid * _ROWS_PER_W
    pltpu.sync_copy(x_hbm.at[pl.ds(base, _ROWS_PER_W)], idx_v)

    bufs = (buf0, buf1)
    gsems = (g0, g1)
    osems = (o0, o1)

    def gather_desc(i, b):
        return pltpu.make_async_copy(
            tab_hbm.at[idx_v.at[pl.ds(i * _CHUNK, _CHUNK)]], bufs[b], gsems[b]
        )

    def write_desc(i, b):
        return pltpu.make_async_copy(
            bufs[b], out_hbm.at[pl.ds(base + i * _CHUNK, _CHUNK)], osems[b]
        )

    gather_desc(0, 0).start()
    gather_desc(1, 1).start()

    def body(j, carry):
        for b in range(2):
            i = 2 * j + b
            gather_desc(i, b).wait()
            write_desc(i, b).start()
            write_desc(i, b).wait()
            gather_desc(i + 2, b).start()
        return carry

    lax.fori_loop(0, _N_CHUNKS // 2 - 1, body, 0, unroll=False)

    for b in range(2):
        i = _N_CHUNKS - 2 + b
        gather_desc(i, b).wait()
        write_desc(i, b).start()
    for b in range(2):
        write_desc(_N_CHUNKS - 2 + b, b).wait()


def kernel(x, embed_tokens):
    flat = x.reshape(-1).astype(jnp.int32)
    out = _embed_gather(flat, embed_tokens)
    return out.reshape(x.shape[0], x.shape[1], D_MODEL)
